# baseline (speedup 1.0000x reference)
import functools

import jax
import jax.numpy as jnp
from jax import lax
from jax.experimental import pallas as pl
from jax.experimental.pallas import tpu as pltpu
from jax.experimental.pallas import tpu_sc as plsc

B = 4
L = 4096
D_MODEL = 768
D_PE = D_MODEL // 2
MAX_LEN = 4096

NUM_CORES = 2
NUM_SUBCORES = 16
NUM_WORKERS = NUM_CORES * NUM_SUBCORES

ROWS = B * L * 2
ROWS_PER_WORKER = ROWS // NUM_WORKERS
CHUNK = 64
NCHUNK = ROWS_PER_WORKER // CHUNK
LANE = 16
COLS = D_PE // LANE


def _pe_add_kernel(f_hbm, idx_hbm, table_hbm, out_hbm,
                   idx_v, f_buf, pe_buf, gsem):
    wid = lax.axis_index("s") * NUM_CORES + lax.axis_index("c")
    worker_base = wid * ROWS_PER_WORKER
    for c in range(NCHUNK):
        base = worker_base + c * CHUNK
        pltpu.sync_copy(idx_hbm.at[pl.ds(base, CHUNK)], idx_v)
        gather = pltpu.async_copy(table_hbm.at[idx_v], pe_buf, gsem)
        pltpu.sync_copy(f_hbm.at[pl.ds(base, CHUNK)], f_buf)
        gather.wait()

        @plsc.parallel_loop(0, CHUNK, 1, unroll=2)
        def _add_row(r):
            for k in range(COLS):
                sl = pl.ds(k * LANE, LANE)
                plsc.addupdate(pe_buf.at[r, sl], f_buf[r, sl])

        pltpu.sync_copy(pe_buf, out_hbm.at[pl.ds(base, CHUNK)])


@jax.jit
def _pe_add(f_rows, idx, table):
    mesh = plsc.VectorSubcoreMesh(core_axis_name="c", subcore_axis_name="s")
    return pl.kernel(
        _pe_add_kernel,
        out_type=jax.ShapeDtypeStruct((ROWS, D_PE), jnp.float32),
        mesh=mesh,
        scratch_types=[
            pltpu.VMEM((CHUNK,), jnp.int32),
            pltpu.VMEM((CHUNK, D_PE), jnp.float32),
            pltpu.VMEM((CHUNK, D_PE), jnp.float32),
            pltpu.SemaphoreType.DMA,
        ],
    )(f_rows, idx, table)


def kernel(f, rank, pe):
    f_rows = f.reshape(ROWS, D_PE)
    idx = rank.astype(jnp.int32).reshape(ROWS)
    table = pe.reshape(MAX_LEN, D_PE)
    out = _pe_add(f_rows, idx, table)
    return out.reshape(B, L, D_MODEL)


# speedup vs baseline: 1.1985x; 1.1985x over previous
import functools

import jax
import jax.numpy as jnp
from jax import lax
from jax.experimental import pallas as pl
from jax.experimental.pallas import tpu as pltpu
from jax.experimental.pallas import tpu_sc as plsc

B = 4
L = 4096
D_MODEL = 768
D_PE = D_MODEL // 2
MAX_LEN = 4096

NUM_CORES = 2
NUM_SUBCORES = 16
NUM_WORKERS = NUM_CORES * NUM_SUBCORES

ROWS = B * L * 2
ROWS_PER_WORKER = ROWS // NUM_WORKERS
CHUNK = 64
NCHUNK = ROWS_PER_WORKER // CHUNK
LANE = 16
COLS = D_PE // LANE
NSLOT = 2


def _pe_add_kernel(f_hbm, idx_hbm, table_hbm, out_hbm,
                   idx_v, f_buf0, f_buf1, pe_buf0, pe_buf1,
                   gsem0, gsem1, fsem0, fsem1, ssem0, ssem1):
    f_bufs = (f_buf0, f_buf1)
    pe_bufs = (pe_buf0, pe_buf1)
    gsems = (gsem0, gsem1)
    fsems = (fsem0, fsem1)
    ssems = (ssem0, ssem1)

    wid = lax.axis_index("s") * NUM_CORES + lax.axis_index("c")
    base = wid * ROWS_PER_WORKER
    pltpu.sync_copy(idx_hbm.at[wid], idx_v)

    loads = [None] * NSLOT
    stores = [None] * NSLOT
    for c in range(NCHUNK + 1):
        if c < NCHUNK:
            s = c % NSLOT
            if stores[s] is not None:
                stores[s].wait()
                stores[s] = None
            g = pltpu.async_copy(table_hbm.at[idx_v.at[c]], pe_bufs[s],
                                 gsems[s])
            fd = pltpu.async_copy(f_hbm.at[pl.ds(base + c * CHUNK, CHUNK)],
                                  f_bufs[s], fsems[s])
            loads[s] = (g, fd)
        if c >= 1:
            p = (c - 1) % NSLOT
            g, fd = loads[p]
            g.wait()
            fd.wait()

            @plsc.parallel_loop(0, CHUNK, 1, unroll=2)
            def _add_row(r):
                for k in range(COLS):
                    sl = pl.ds(k * LANE, LANE)
                    plsc.addupdate(pe_bufs[p].at[r, sl], f_bufs[p][r, sl])

            stores[p] = pltpu.async_copy(
                pe_bufs[p],
                out_hbm.at[pl.ds(base + (c - 1) * CHUNK, CHUNK)],
                ssems[p])
    for s in range(NSLOT):
        if stores[s] is not None:
            stores[s].wait()


@jax.jit
def _pe_add(f_rows, idx, table):
    mesh = plsc.VectorSubcoreMesh(core_axis_name="c", subcore_axis_name="s")
    return pl.kernel(
        _pe_add_kernel,
        out_type=jax.ShapeDtypeStruct((ROWS, D_PE), jnp.float32),
        mesh=mesh,
        scratch_types=[
            pltpu.VMEM((NCHUNK, CHUNK), jnp.int32),
            pltpu.VMEM((CHUNK, D_PE), jnp.float32),
            pltpu.VMEM((CHUNK, D_PE), jnp.float32),
            pltpu.VMEM((CHUNK, D_PE), jnp.float32),
            pltpu.VMEM((CHUNK, D_PE), jnp.float32),
            pltpu.SemaphoreType.DMA,
            pltpu.SemaphoreType.DMA,
            pltpu.SemaphoreType.DMA,
            pltpu.SemaphoreType.DMA,
            pltpu.SemaphoreType.DMA,
            pltpu.SemaphoreType.DMA,
        ],
    )(f_rows, idx, table)


def kernel(f, rank, pe):
    f_rows = f.reshape(ROWS, D_PE)
    idx = rank.astype(jnp.int32).reshape(NUM_WORKERS, NCHUNK, CHUNK)
    table = pe.reshape(MAX_LEN, D_PE)
    out = _pe_add(f_rows, idx, table)
    return out.reshape(B, L, D_MODEL)


# speedup vs baseline: 2.6192x; 2.1854x over previous
import functools

import jax
import jax.numpy as jnp
from jax import lax
from jax.experimental import pallas as pl
from jax.experimental.pallas import tpu as pltpu
from jax.experimental.pallas import tpu_sc as plsc

B = 4
L = 4096
D_MODEL = 768
D_PE = D_MODEL // 2
MAX_LEN = 4096

NUM_CORES = 2
NUM_SUBCORES = 16
NUM_WORKERS = NUM_CORES * NUM_SUBCORES

POS = B * L
POS_PER_WORKER = POS // NUM_WORKERS
WORKERS_PER_BATCH = NUM_WORKERS // B
CHUNK = 16
NCHUNK = POS_PER_WORKER // CHUNK
GROWS = 2 * CHUNK
LANE = 16
COLS = D_PE // LANE
NSLOT = 4


def _pe_add_kernel(f_hbm, idx_hbm, table_hbm, out_hbm,
                   idx_v, f_buf0, f_buf1, f_buf2, f_buf3,
                   pe_buf0, pe_buf1, pe_buf2, pe_buf3,
                   gsem0, gsem1, gsem2, gsem3,
                   fsem0, fsem1, fsem2, fsem3,
                   ssem0, ssem1, ssem2, ssem3):
    f_bufs = (f_buf0, f_buf1, f_buf2, f_buf3)
    pe_bufs = (pe_buf0, pe_buf1, pe_buf2, pe_buf3)
    gsems = (gsem0, gsem1, gsem2, gsem3)
    fsems = (fsem0, fsem1, fsem2, fsem3)
    ssems = (ssem0, ssem1, ssem2, ssem3)

    wid = lax.axis_index("s") * NUM_CORES + lax.axis_index("c")
    b = wid // WORKERS_PER_BATCH
    l0 = (wid % WORKERS_PER_BATCH) * POS_PER_WORKER
    pltpu.sync_copy(idx_hbm.at[wid], idx_v)

    def f_slice(c):
        return f_hbm.at[b, pl.ds(l0 + c * CHUNK, CHUNK), :]

    def out_slice(c):
        return out_hbm.at[b, pl.ds(l0 + c * CHUNK, CHUNK), :]

    def issue_loads(c, s):
        pltpu.async_copy(table_hbm.at[idx_v.at[c]], pe_bufs[s], gsems[s])
        pltpu.async_copy(f_slice(c), f_bufs[s], fsems[s])

    def wait_loads(c, s):
        pltpu.make_async_copy(table_hbm.at[idx_v.at[c]], pe_bufs[s],
                              gsems[s]).wait()
        pltpu.make_async_copy(f_slice(c), f_bufs[s], fsems[s]).wait()

    def do_add(s):
        @plsc.parallel_loop(0, CHUNK, 1, unroll=1)
        def _add_pos(r):
            for half in range(2):
                for k in range(COLS):
                    plsc.addupdate(
                        f_bufs[s].at[r, pl.ds(half * D_PE + k * LANE, LANE)],
                        pe_bufs[s][2 * r + half, pl.ds(k * LANE, LANE)])

    def issue_store(c, s):
        pltpu.async_copy(f_bufs[s], out_slice(c), ssems[s])

    def wait_store(c, s):
        pltpu.make_async_copy(f_bufs[s], out_slice(c), ssems[s]).wait()

    issue_loads(0, 0)
    issue_loads(1, 1)
    issue_loads(2, 2)
    wait_loads(0, 0)
    do_add(0)
    issue_store(0, 0)
    issue_loads(3, 3)
    wait_loads(1, 1)
    do_add(1)
    issue_store(1, 1)

    @pl.loop(0, NCHUNK - 4, step=NSLOT)
    def _grp(i):
        for s in range(NSLOT):
            t = i + 4 + s
            wait_store(t - 4, s)
            issue_loads(t, s)
            p = (s + 2) % NSLOT
            wait_loads(t - 2, p)
            do_add(p)
            issue_store(t - 2, p)

    wait_loads(NCHUNK - 2, (NCHUNK - 2) % NSLOT)
    do_add((NCHUNK - 2) % NSLOT)
    issue_store(NCHUNK - 2, (NCHUNK - 2) % NSLOT)
    wait_loads(NCHUNK - 1, (NCHUNK - 1) % NSLOT)
    do_add((NCHUNK - 1) % NSLOT)
    issue_store(NCHUNK - 1, (NCHUNK - 1) % NSLOT)
    for s in range(NSLOT):
        wait_store(NCHUNK - 4 + s, (NCHUNK - 4 + s) % NSLOT)


@jax.jit
def _pe_add(f, idx, table):
    mesh = plsc.VectorSubcoreMesh(core_axis_name="c", subcore_axis_name="s")
    return pl.kernel(
        _pe_add_kernel,
        out_type=jax.ShapeDtypeStruct((B, L, D_MODEL), jnp.float32),
        mesh=mesh,
        scratch_types=(
            [pltpu.VMEM((NCHUNK, GROWS), jnp.int32)]
            + [pltpu.VMEM((CHUNK, D_MODEL), jnp.float32)] * NSLOT
            + [pltpu.VMEM((GROWS, D_PE), jnp.float32)] * NSLOT
            + [pltpu.SemaphoreType.DMA] * (3 * NSLOT)
        ),
    )(f, idx, table)


def kernel(f, rank, pe):
    idx = rank.astype(jnp.int32).reshape(NUM_WORKERS, NCHUNK, GROWS)
    table = pe.reshape(MAX_LEN, D_PE)
    return _pe_add(f, idx, table)
